# baseline (speedup 1.0000x reference)
import jax
import jax.numpy as jnp
from jax import lax
from jax.experimental import pallas as pl
from jax.experimental.pallas import tpu as pltpu

H, W, N = 512, 1024, 48
NUM_STUFF = 11
IGNORE = 255
BH = 64
PH1 = 24
CH = 8
N_TAIL_CHUNKS = (N - PH1) // CH


def _body(seg_ref, mask_ref, mask_any, out_ref, best_ref, buf_ref, sem):
    ib = pl.program_id(0)
    m = mask_ref[...]
    w1 = (N - PH1 + NUM_STUFF
          + lax.broadcasted_iota(jnp.int32, (PH1, 1, 1), 0)).astype(jnp.float32)
    best = jnp.max(m * w1, axis=0)
    best_ref[...] = best

    def cond(carry):
        c, done = carry
        return (c >= 0) & jnp.logical_not(done)

    def body(carry):
        c, _ = carry
        cp = pltpu.make_async_copy(
            mask_any.at[pl.ds(c * CH, CH), pl.ds(ib * BH, BH), :], buf_ref, sem)
        cp.start()
        cp.wait()
        w = (c * CH + NUM_STUFF
             + lax.broadcasted_iota(jnp.int32, (CH, 1, 1), 0)).astype(jnp.float32)
        nb = jnp.maximum(best_ref[...], jnp.max(buf_ref[...] * w, axis=0))
        best_ref[...] = nb
        return c - 1, jnp.min(nb) > 0

    lax.while_loop(cond, body, (N_TAIL_CHUNKS - 1, jnp.min(best) > 0))

    seg = seg_ref[0]
    fallback = jnp.where(seg <= NUM_STUFF - 1, seg, IGNORE)
    bestf = best_ref[...]
    out_ref[0] = jnp.where(bestf > 0, bestf.astype(jnp.int32), fallback)


def kernel(gt_segs, gt_masks):
    grid = (H // BH,)
    return pl.pallas_call(
        _body,
        grid=grid,
        in_specs=[
            pl.BlockSpec((1, BH, W), lambda i: (0, i, 0)),
            pl.BlockSpec((PH1, BH, W), lambda i: ((N - PH1) // PH1, i, 0)),
            pl.BlockSpec(memory_space=pl.MemorySpace.ANY),
        ],
        out_specs=pl.BlockSpec((1, BH, W), lambda i: (0, i, 0)),
        out_shape=jax.ShapeDtypeStruct((1, H, W), jnp.int32),
        scratch_shapes=[
            pltpu.VMEM((BH, W), jnp.float32),
            pltpu.VMEM((CH, BH, W), jnp.float32),
            pltpu.SemaphoreType.DMA,
        ],
    )(gt_segs, gt_masks, gt_masks)


# speedup vs baseline: 1.0314x; 1.0314x over previous
import jax
import jax.numpy as jnp
from jax import lax
from jax.experimental import pallas as pl
from jax.experimental.pallas import tpu as pltpu

H, W, N = 512, 1024, 48
NUM_STUFF = 11
IGNORE = 255
BH = 64
PH1A = 4
PH1B = 16
PH1 = PH1A + PH1B
CH = 4
N_TAIL_CHUNKS = (N - PH1) // CH


def _body(seg_ref, mask_a, mask_b, mask_any, out_ref, best_ref, buf_ref, sem):
    ib = pl.program_id(0)
    wa = (N - PH1 + NUM_STUFF
          + lax.broadcasted_iota(jnp.int32, (PH1A, 1, 1), 0)).astype(jnp.float32)
    wb = (N - PH1B + NUM_STUFF
          + lax.broadcasted_iota(jnp.int32, (PH1B, 1, 1), 0)).astype(jnp.float32)
    best = jnp.maximum(jnp.max(mask_a[...] * wa, axis=0),
                       jnp.max(mask_b[...] * wb, axis=0))
    best_ref[...] = best

    def cond(carry):
        c, done = carry
        return (c >= 0) & jnp.logical_not(done)

    def body(carry):
        c, _ = carry
        cp = pltpu.make_async_copy(
            mask_any.at[pl.ds(c * CH, CH), pl.ds(ib * BH, BH), :], buf_ref, sem)
        cp.start()
        cp.wait()
        w = (c * CH + NUM_STUFF
             + lax.broadcasted_iota(jnp.int32, (CH, 1, 1), 0)).astype(jnp.float32)
        nb = jnp.maximum(best_ref[...], jnp.max(buf_ref[...] * w, axis=0))
        best_ref[...] = nb
        return c - 1, jnp.min(nb) > 0

    lax.while_loop(cond, body, (N_TAIL_CHUNKS - 1, jnp.min(best) > 0))

    seg = seg_ref[0]
    fallback = jnp.where(seg <= NUM_STUFF - 1, seg, IGNORE)
    bestf = best_ref[...]
    out_ref[0] = jnp.where(bestf > 0, bestf.astype(jnp.int32), fallback)


def kernel(gt_segs, gt_masks):
    grid = (H // BH,)
    return pl.pallas_call(
        _body,
        grid=grid,
        in_specs=[
            pl.BlockSpec((1, BH, W), lambda i: (0, i, 0)),
            pl.BlockSpec((PH1A, BH, W), lambda i: ((N - PH1) // PH1A, i, 0)),
            pl.BlockSpec((PH1B, BH, W), lambda i: ((N - PH1B) // PH1B, i, 0)),
            pl.BlockSpec(memory_space=pl.MemorySpace.ANY),
        ],
        out_specs=pl.BlockSpec((1, BH, W), lambda i: (0, i, 0)),
        out_shape=jax.ShapeDtypeStruct((1, H, W), jnp.int32),
        scratch_shapes=[
            pltpu.VMEM((BH, W), jnp.float32),
            pltpu.VMEM((CH, BH, W), jnp.float32),
            pltpu.SemaphoreType.DMA,
        ],
    )(gt_segs, gt_masks, gt_masks, gt_masks)
